# baseline (speedup 1.0000x reference)
import functools

import jax
import jax.numpy as jnp
from jax import lax
from jax.experimental import pallas as pl
from jax.experimental.pallas import tpu as pltpu
from jax.experimental.pallas import tpu_sc as plsc

B = 4096
D = 128


def _sc_gather6(tables, idx3):
    info = plsc.get_sparse_core_info()
    nw = info.num_cores * info.num_subcores
    bpw = B // nw
    nbuf = 6
    nt = 6
    mesh = plsc.VectorSubcoreMesh(core_axis_name="c", subcore_axis_name="s")
    out_t = tuple(jax.ShapeDtypeStruct((B, D), jnp.float32) for _ in range(nt))

    idx_all = idx3.reshape(3, nw, bpw).transpose(1, 0, 2)

    @functools.partial(
        pl.kernel, mesh=mesh, out_type=out_t,
        scratch_types=[
            pltpu.VMEM((3, bpw), jnp.int32),
            pltpu.VMEM((nbuf, bpw, D), jnp.float32),
            pltpu.SemaphoreType.DMA((nbuf,)),
            pltpu.SemaphoreType.DMA((nbuf,)),
        ],
    )
    def k(t0, t1, t2, t3, t4, t5, idx_hbm, o0, o1, o2, o3, o4, o5,
          idx_v, rows, gsem, ssem):
        wid = lax.axis_index("s") * info.num_cores + lax.axis_index("c")
        base = wid * bpw
        pltpu.sync_copy(idx_hbm.at[wid], idx_v)
        tabs = [t0, t1, t2, t3, t4, t5]
        outs = [o0, o1, o2, o3, o4, o5]
        g = [None] * nt
        s = [None] * nt

        def launch_scatter(kk):
            b = kk % nbuf
            g[kk].wait()
            s[kk] = pltpu.async_copy(
                rows.at[b], outs[kk].at[pl.ds(base, bpw)], ssem.at[b])

        for t in range(nt):
            b = t % nbuf
            if t >= nbuf:
                s[t - nbuf].wait()
            g[t] = pltpu.async_copy(tabs[t].at[idx_v.at[t // 2]], rows.at[b],
                                    gsem.at[b])
            if t >= nbuf - 1:
                launch_scatter(t - (nbuf - 1))
        for kk in range(max(nt - (nbuf - 1), 0), nt):
            launch_scatter(kk)
        for kk in range(max(nt - nbuf, 0), nt):
            s[kk].wait()

    return k(*tables, idx_all)


def _sc_score(entity_re, entity_im, relation_re, relation_im, idx3):
    info = plsc.get_sparse_core_info()
    nw = info.num_cores * info.num_subcores
    bpw = B // nw
    half = bpw // 2
    mesh = plsc.VectorSubcoreMesh(core_axis_name="c", subcore_axis_name="s")

    idx_all = idx3.reshape(3, nw, bpw).transpose(1, 0, 2)

    gdn = lax.GatherDimensionNumbers(offset_dims=(), collapsed_slice_dims=(0,),
                                     start_index_map=(0,))

    def take16(x, p):
        return lax.gather(x, p, gdn, slice_sizes=(1,),
                          mode=lax.GatherScatterMode.PROMISE_IN_BOUNDS)

    @functools.partial(
        pl.kernel, mesh=mesh,
        out_type=jax.ShapeDtypeStruct((B // 16, 16), jnp.float32),
        scratch_types=[
            pltpu.VMEM((3, bpw), jnp.int32),
            [pltpu.VMEM((bpw, D), jnp.float32) for _ in range(6)],
            pltpu.VMEM((bpw // 16, 16), jnp.float32),
            pltpu.SemaphoreType.DMA((12,)),
        ],
    )
    def k(ent_re, ent_im, rel_re, rel_im, idx_hbm, out, idx_v, ops, score_v,
          gsem):
        wid = lax.axis_index("s") * info.num_cores + lax.axis_index("c")
        pltpu.sync_copy(idx_hbm.at[wid], idx_v)
        lane = lax.iota(jnp.int32, 16)
        perms = [(lane ^ (1 << s)).reshape(16, 1) for s in range(4)]
        masks = [lane == r for r in range(16)]
        tabs = [ent_re, ent_im, ent_re, ent_im, rel_re, rel_im]
        descs = []
        for h in range(2):
            for i in range(6):
                descs.append(pltpu.async_copy(
                    tabs[i].at[idx_v.at[i // 2, pl.ds(h * half, half)]],
                    ops[i].at[pl.ds(h * half, half)],
                    gsem.at[h * 6 + i]))

        def grp_body(g, u):
            g0 = g * 16
            merged = jnp.zeros((16,), jnp.float32)
            for r in range(16):
                row = g0 + r
                acc = jnp.zeros((16,), jnp.float32)
                for c in range(8):
                    sl = pl.ds(c * 16, 16)
                    hre = ops[0][row, sl]
                    him = ops[1][row, sl]
                    tre = ops[2][row, sl]
                    tim = ops[3][row, sl]
                    rre = ops[4][row, sl]
                    rim = ops[5][row, sl]
                    acc = (acc + hre * (rre * tre + rim * tim)
                           + him * (rre * tim - rim * tre))
                for p in perms:
                    acc = acc + take16(acc, p)
                merged = jnp.where(masks[r], acc, merged)
            score_v[g, :] = 1.0 / (1.0 + jnp.exp(-merged))
            return u

        for h in range(2):
            for i in range(6):
                descs[h * 6 + i].wait()
            lax.fori_loop(h * 4, h * 4 + 4, grp_body, 0)
        pltpu.sync_copy(score_v, out.at[pl.ds(wid * (bpw // 16), bpw // 16)])

    return k(entity_re, entity_im, relation_re, relation_im, idx_all)


def _tc_gram(ure, uim, ire, iim, bre, bim, reg_ref):
    def gram_norm(a_ref, b_ref):
        a = a_ref[...]
        b = b_ref[...]
        dn = (((0,), (0,)), ((), ()))
        g = (lax.dot_general(a, a, dn, preferred_element_type=jnp.float32)
             + lax.dot_general(b, b, dn, preferred_element_type=jnp.float32))
        return jnp.sqrt(jnp.sum(g * g))

    reg = gram_norm(ure, uim) + gram_norm(ire, iim) + gram_norm(bre, bim)
    reg_ref[...] = reg.reshape(1, 1)


def kernel(entity_re, entity_im, relation_re, relation_im,
           head, tail, relation, reg_user, reg_item, reg_brand):
    idx_reg = jnp.stack([reg_user, reg_item, reg_brand])
    idx_score = jnp.stack([head, tail, relation])
    ure, uim, ire, iim, bre, bim = _sc_gather6(
        [entity_re, entity_im, entity_re, entity_im, entity_re, entity_im],
        idx_reg)
    score2d = _sc_score(entity_re, entity_im, relation_re, relation_im,
                        idx_score)
    reg = pl.pallas_call(
        _tc_gram,
        out_shape=jax.ShapeDtypeStruct((1, 1), jnp.float32),
    )(ure, uim, ire, iim, bre, bim)
    return score2d.reshape(B), reg[0, 0]


# speedup vs baseline: 1.1698x; 1.1698x over previous
import functools

import jax
import jax.numpy as jnp
from jax import lax
from jax.experimental import pallas as pl
from jax.experimental.pallas import tpu as pltpu
from jax.experimental.pallas import tpu_sc as plsc

B = 4096
D = 128


def _sc_main(entity_re, entity_im, relation_re, relation_im, idx6):
    info = plsc.get_sparse_core_info()
    nw = info.num_cores * info.num_subcores
    bpw = B // nw
    qrt = bpw // 4
    nbuf = 3
    mesh = plsc.VectorSubcoreMesh(core_axis_name="c", subcore_axis_name="s")

    idx_all = idx6.reshape(6, nw, bpw).transpose(1, 0, 2)

    out_t = (jax.ShapeDtypeStruct((B, 16), jnp.float32),) + tuple(
        jax.ShapeDtypeStruct((B, D), jnp.float32) for _ in range(6))

    @functools.partial(
        pl.kernel, mesh=mesh, out_type=out_t,
        scratch_types=[
            pltpu.VMEM((6, bpw), jnp.int32),
            [pltpu.VMEM((bpw, D), jnp.float32) for _ in range(6)],
            pltpu.VMEM((nbuf, qrt, D), jnp.float32),
            pltpu.VMEM((bpw, 16), jnp.float32),
            pltpu.SemaphoreType.DMA((6,)),
            pltpu.SemaphoreType.DMA((nbuf,)),
            pltpu.SemaphoreType.DMA((nbuf,)),
            pltpu.SemaphoreType.DMA,
        ],
    )
    def k(ent_re, ent_im, rel_re, rel_im, idx_hbm,
          o_acc, o_ure, o_uim, o_ire, o_iim, o_bre, o_bim,
          idx_v, ops, rbuf, acc_buf, qsem, gsem, ssem, osem):
        wid = lax.axis_index("s") * info.num_cores + lax.axis_index("c")
        base = wid * bpw
        pltpu.sync_copy(idx_hbm.at[wid], idx_v)

        score_tabs = [ent_re, ent_im, ent_re, ent_im, rel_re, rel_im]
        sdescs = [
            pltpu.async_copy(score_tabs[i].at[idx_v.at[i // 2]], ops[i],
                             qsem.at[i])
            for i in range(6)
        ]

        reg_plan = [
            (3, ent_re, o_ure), (3, ent_im, o_uim),
            (4, ent_re, o_ire), (4, ent_im, o_iim),
            (5, ent_re, o_bre), (5, ent_im, o_bim),
        ]
        steps = []
        for j, tab, out in reg_plan:
            for h in range(4):
                steps.append((j, h, tab, out))
        ns = len(steps)
        g = [None] * ns
        s = [None] * ns

        def ring_gather(i):
            j, h, tab, _ = steps[i]
            g[i] = pltpu.async_copy(
                tab.at[idx_v.at[j, pl.ds(h * qrt, qrt)]],
                rbuf.at[i % nbuf], gsem.at[i % nbuf])

        def ring_scatter(i):
            _, h, _, out = steps[i]
            g[i].wait()
            s[i] = pltpu.async_copy(
                rbuf.at[i % nbuf],
                out.at[pl.ds(base + h * qrt, qrt)], ssem.at[i % nbuf])

        def row_body(r, u, lo):
            row = lo + r
            acc = jnp.zeros((16,), jnp.float32)
            for c in range(8):
                sl = pl.ds(c * 16, 16)
                hre = ops[0][row, sl]
                him = ops[1][row, sl]
                tre = ops[2][row, sl]
                tim = ops[3][row, sl]
                rre = ops[4][row, sl]
                rim = ops[5][row, sl]
                acc = (acc + hre * (rre * tre + rim * tim)
                       + him * (rre * tim - rim * tre))
            acc_buf[row, :] = acc
            return u

        grp = 0
        for i in range(ns):
            if i >= nbuf:
                s[i - nbuf].wait()
            ring_gather(i)
            if i >= 2:
                ring_scatter(i - 2)
            if i >= 16 and grp < 8:
                if grp == 0:
                    for d_ in sdescs:
                        d_.wait()
                lax.fori_loop(0, 16,
                              functools.partial(row_body, lo=grp * 16), 0)
                grp += 1
        for i in range(ns - 2, ns):
            ring_scatter(i)
        while grp < 8:
            if grp == 0:
                for d_ in sdescs:
                    d_.wait()
            lax.fori_loop(0, 16, functools.partial(row_body, lo=grp * 16), 0)
            grp += 1
        od = pltpu.async_copy(acc_buf, o_acc.at[pl.ds(base, bpw)], osem)
        for i in range(ns - nbuf, ns):
            s[i].wait()
        od.wait()

    return k(entity_re, entity_im, relation_re, relation_im, idx_all)


def _tc_finish(ure, uim, ire, iim, bre, bim, acc, score_ref, reg_ref):
    score_ref[...] = jax.nn.sigmoid(jnp.sum(acc[...], axis=1))

    def gram_norm(a_ref, b_ref):
        a = a_ref[...]
        b = b_ref[...]
        dn = (((0,), (0,)), ((), ()))
        g = (lax.dot_general(a, a, dn, preferred_element_type=jnp.float32)
             + lax.dot_general(b, b, dn, preferred_element_type=jnp.float32))
        return jnp.sqrt(jnp.sum(g * g))

    reg = gram_norm(ure, uim) + gram_norm(ire, iim) + gram_norm(bre, bim)
    reg_ref[...] = reg.reshape(1, 1)


def kernel(entity_re, entity_im, relation_re, relation_im,
           head, tail, relation, reg_user, reg_item, reg_brand):
    idx6 = jnp.stack([head, tail, relation, reg_user, reg_item, reg_brand])
    acc, ure, uim, ire, iim, bre, bim = _sc_main(
        entity_re, entity_im, relation_re, relation_im, idx6)
    score, reg = pl.pallas_call(
        _tc_finish,
        out_shape=(jax.ShapeDtypeStruct((B,), jnp.float32),
                   jax.ShapeDtypeStruct((1, 1), jnp.float32)),
    )(ure, uim, ire, iim, bre, bim, acc)
    return score, reg[0, 0]
